# baseline (speedup 1.0000x reference)
import functools

import jax
import jax.numpy as jnp
from jax import lax
from jax.experimental import pallas as pl
from jax.experimental.pallas import tpu as pltpu
from jax.experimental.pallas import tpu_sc as plsc

NC = 2
NS = 16
NW = NC * NS
CH = 128

_HI = lax.Precision.HIGHEST


def _sc_mesh():
    return plsc.VectorSubcoreMesh(
        core_axis_name="c", subcore_axis_name="s", num_cores=NC, num_subcores=NS
    )


def _degree_call(src2d, dst2d, ones_s, ones_d, zeros_r, n_pad, n_chunk):
    rt = n_pad // NS

    @functools.partial(
        pl.kernel,
        out_type=jax.ShapeDtypeStruct((NC, n_pad, 2), jnp.float32),
        mesh=_sc_mesh(),
        scratch_types=[
            pltpu.VMEM((n_chunk, CH), jnp.int32),
            pltpu.VMEM((n_chunk, CH), jnp.int32),
            pltpu.VMEM((CH, 2), jnp.float32),
            pltpu.VMEM((CH, 2), jnp.float32),
            pltpu.VMEM((rt, 2), jnp.float32),
            pltpu.VMEM_SHARED((n_pad, 2), jnp.float32),
        ],
        compiler_params=pltpu.CompilerParams(use_tc_tiling_on_sc=False),
    )
    def k(src_h, dst_h, os_h, od_h, zer_h, out_h, idx_s, idx_d, os_v, od_v,
          buf, deg):
        c = lax.axis_index("c")
        s = lax.axis_index("s")
        wid = c * NS + s
        pltpu.sync_copy(src_h.at[pl.ds(wid * n_chunk, n_chunk)], idx_s)
        pltpu.sync_copy(dst_h.at[pl.ds(wid * n_chunk, n_chunk)], idx_d)
        pltpu.sync_copy(os_h, os_v)
        pltpu.sync_copy(od_h, od_v)
        pltpu.sync_copy(zer_h, buf)
        pltpu.sync_copy(buf, deg.at[pl.ds(s * rt, rt)])
        plsc.subcore_barrier()

        def body(j, carry):
            pltpu.sync_copy(os_v, deg.at[idx_s.at[j]], add=True)
            pltpu.sync_copy(od_v, deg.at[idx_d.at[j]], add=True)
            return carry

        lax.fori_loop(0, n_chunk, body, 0)
        plsc.subcore_barrier()
        pltpu.sync_copy(deg.at[pl.ds(s * rt, rt)], buf)
        pltpu.sync_copy(buf, out_h.at[c, pl.ds(s * rt, rt)])

    return k(src2d, dst2d, ones_s, ones_d, zeros_r)


def _scatter_call(p, src2d, dst2d, zeros_ch, n_pad, n_chunk, h):
    rt = n_pad // NS
    nz = rt // CH

    nslot = 4
    nhalf = 2
    qc = n_chunk // nhalf
    ngrp = qc // nslot

    @functools.partial(
        pl.kernel,
        out_type=jax.ShapeDtypeStruct((NC, n_pad, h), jnp.float32),
        mesh=_sc_mesh(),
        scratch_types=[
            pltpu.VMEM((qc, CH), jnp.int32),
            pltpu.VMEM((qc, CH), jnp.int32),
        ]
        + [pltpu.VMEM((CH, h), jnp.float32) for _ in range(nslot)]
        + [pltpu.VMEM_SHARED((n_pad, h), jnp.float32)]
        + [pltpu.VMEM_SHARED((n_pad, h), jnp.float32)]
        + [pltpu.SemaphoreType.DMA for _ in range(nslot)],
        compiler_params=pltpu.CompilerParams(use_tc_tiling_on_sc=False),
    )
    def k(p_h, src_h, dst_h, zer_h, out_h, idx_s, idx_d, *rest):
        rows = rest[:nslot]
        buf = rows[0]
        agg = rest[nslot]
        pspm = rest[nslot + 1]
        sems = rest[nslot + 2:]
        c = lax.axis_index("c")
        s = lax.axis_index("s")
        wid = c * NS + s
        for t in range(nz):
            pltpu.sync_copy(p_h.at[pl.ds(s * rt + t * CH, CH)], buf)
            pltpu.sync_copy(buf, pspm.at[pl.ds(s * rt + t * CH, CH)])
        pltpu.sync_copy(zer_h, buf)
        for t in range(nz):
            pltpu.sync_copy(buf, agg.at[pl.ds(s * rt + t * CH, CH)])
        plsc.subcore_barrier()

        for q in range(nhalf):
            base = wid * n_chunk + q * qc
            pltpu.sync_copy(src_h.at[pl.ds(base, qc)], idx_s)
            pltpu.sync_copy(dst_h.at[pl.ds(base, qc)], idx_d)
            for kk in range(nslot):
                pltpu.async_copy(pspm.at[idx_s.at[kk]], rows[kk], sems[kk])

            def body(g, carry):
                for kk in range(nslot):
                    j = g * nslot + kk
                    pltpu.make_async_copy(pspm.at[idx_s.at[j]], rows[kk],
                                          sems[kk]).wait()
                    pltpu.sync_copy(rows[kk], agg.at[idx_d.at[j]], add=True)

                    @pl.when(g < ngrp - 1)
                    def _():
                        pltpu.async_copy(pspm.at[idx_s.at[j + nslot]],
                                         rows[kk], sems[kk])

                return carry

            lax.fori_loop(0, ngrp, body, 0)
        plsc.subcore_barrier()
        for t in range(nz):
            pltpu.sync_copy(agg.at[pl.ds(s * rt + t * CH, CH)], buf)
            pltpu.sync_copy(buf, out_h.at[c, pl.ds(s * rt + t * CH, CH)])

    return k(p, src2d, dst2d, zeros_ch)


def _proj_kernel(f_ref, w_ref, o_ref):
    o_ref[...] = lax.dot_general(f_ref[...], w_ref[...],
                                 (((1,), (0,)), ((), ())),
                                 preferred_element_type=jnp.float32)


def _proj_call(f_pad, w1, n_pad, bm):
    d_in, h = w1.shape
    return pl.pallas_call(
        _proj_kernel,
        grid=(n_pad // bm,),
        in_specs=[
            pl.BlockSpec((bm, d_in), lambda i: (i, 0)),
            pl.BlockSpec((d_in, h), lambda i: (0, 0)),
        ],
        out_specs=pl.BlockSpec((bm, h), lambda i: (i, 0)),
        out_shape=jax.ShapeDtypeStruct((n_pad, h), jnp.float32),
    )(f_pad, w1)


def _normscale_kernel(deg_ref, q_ref, p_ref, ns_ref, nd_ref):
    d = deg_ref[0] + deg_ref[1]
    ns = lax.rsqrt(jnp.maximum(d[:, 0:1], 1.0))
    nd = lax.rsqrt(jnp.maximum(d[:, 1:2], 1.0))
    ns_ref[...] = ns
    nd_ref[...] = nd
    p_ref[...] = q_ref[...] * ns


def _normscale_call(deg, q1, n_pad, bm):
    h = q1.shape[-1]
    return pl.pallas_call(
        _normscale_kernel,
        grid=(n_pad // bm,),
        in_specs=[
            pl.BlockSpec((NC, bm, 2), lambda i: (0, i, 0)),
            pl.BlockSpec((bm, h), lambda i: (i, 0)),
        ],
        out_specs=[
            pl.BlockSpec((bm, h), lambda i: (i, 0)),
            pl.BlockSpec((bm, 1), lambda i: (i, 0)),
            pl.BlockSpec((bm, 1), lambda i: (i, 0)),
        ],
        out_shape=[
            jax.ShapeDtypeStruct((n_pad, h), jnp.float32),
            jax.ShapeDtypeStruct((n_pad, 1), jnp.float32),
            jax.ShapeDtypeStruct((n_pad, 1), jnp.float32),
        ],
    )(deg, q1)


def _combine_kernel(a_ref, nd_ref, ns_ref, b_ref, w_ref, o_ref):
    a = a_ref[0] + a_ref[1]
    hval = jnp.maximum(a * nd_ref[...] + b_ref[...], 0.0)
    o_ref[...] = lax.dot_general(hval * ns_ref[...], w_ref[...],
                                 (((1,), (0,)), ((), ())),
                                 preferred_element_type=jnp.float32)


def _combine_call(agg, nd_col, ns_col, b, w, n_pad, bm):
    h_in, h_out = w.shape
    grid = (n_pad // bm,)
    return pl.pallas_call(
        _combine_kernel,
        grid=grid,
        in_specs=[
            pl.BlockSpec((NC, bm, h_in), lambda i: (0, i, 0)),
            pl.BlockSpec((bm, 1), lambda i: (i, 0)),
            pl.BlockSpec((bm, 1), lambda i: (i, 0)),
            pl.BlockSpec((h_in,), lambda i: (0,)),
            pl.BlockSpec((h_in, h_out), lambda i: (0, 0)),
        ],
        out_specs=pl.BlockSpec((bm, h_out), lambda i: (i, 0)),
        out_shape=jax.ShapeDtypeStruct((n_pad, h_out), jnp.float32),
    )(agg, nd_col, ns_col, b, w)


def _zcombine_kernel(a_ref, nd_ref, b_ref, z_ref):
    z_ref[...] = (a_ref[0] + a_ref[1]) * nd_ref[...] + b_ref[...]


def _zcombine_call(agg, nd_col, b3, n, bm):
    h = agg.shape[-1]
    return pl.pallas_call(
        _zcombine_kernel,
        grid=(n // bm,),
        in_specs=[
            pl.BlockSpec((NC, bm, h), lambda i: (0, i, 0)),
            pl.BlockSpec((bm, 1), lambda i: (i, 0)),
            pl.BlockSpec((h,), lambda i: (0,)),
        ],
        out_specs=pl.BlockSpec((bm, h), lambda i: (i, 0)),
        out_shape=jax.ShapeDtypeStruct((n, h), jnp.float32),
    )(agg, nd_col, b3)


def _decoder_kernel(zi_ref, zall_ref, adj_ref):
    t = lax.dot_general(zi_ref[...], zall_ref[...], (((1,), (1,)), ((), ())),
                        preferred_element_type=jnp.float32)
    adj_ref[...] = 0.5 * jnp.tanh(0.5 * t) + 0.5


def _decoder_call(z, n, bd):
    h = z.shape[-1]
    return pl.pallas_call(
        _decoder_kernel,
        grid=(n // bd,),
        in_specs=[
            pl.BlockSpec((bd, h), lambda i: (i, 0)),
            pl.BlockSpec((n, h), lambda i: (0, 0)),
        ],
        out_specs=pl.BlockSpec((bd, n), lambda i: (i, 0)),
        out_shape=jax.ShapeDtypeStruct((n, n), jnp.float32),
    )(z, z)


def kernel(features, edge_index, W1, b1, W2, b2, W3, b3):
    n, _ = features.shape
    e = edge_index.shape[1]
    h1 = W1.shape[1]
    h2 = W3.shape[1]

    n_pad = ((n + 1 + NS * CH - 1) // (NS * CH)) * (NS * CH)
    ept = ((e + NW * CH * 8 - 1) // (NW * CH * 8)) * CH * 8
    ep = ept * NW
    n_chunk = ept // CH
    pad_row = n

    src = edge_index[0]
    dst = edge_index[1]
    pad = jnp.full((ep - e,), pad_row, jnp.int32)
    src2d = jnp.concatenate([src, pad]).reshape(ep // CH, CH)
    dst2d = jnp.concatenate([dst, pad]).reshape(ep // CH, CH)

    f_pad = jnp.pad(features, ((0, n_pad - n), (0, 0)))
    ones_s = jnp.tile(jnp.array([[1.0, 0.0]], jnp.float32), (CH, 1))
    ones_d = jnp.tile(jnp.array([[0.0, 1.0]], jnp.float32), (CH, 1))
    zeros_r = jnp.zeros((n_pad // NS, 2), jnp.float32)
    zeros_h1 = jnp.zeros((CH, h1), jnp.float32)
    zeros_h2 = jnp.zeros((CH, h2), jnp.float32)

    bm = 2048
    deg = _degree_call(src2d, dst2d, ones_s, ones_d, zeros_r, n_pad, n_chunk)
    q1 = _proj_call(f_pad, W1, n_pad, bm)
    p1, ns_col, nd_col = _normscale_call(deg, q1, n_pad, bm)
    agg1 = _scatter_call(p1, src2d, dst2d, zeros_h1, n_pad, n_chunk, h1)
    p2 = _combine_call(agg1, nd_col, ns_col, b1, W2, n_pad, bm)
    agg2 = _scatter_call(p2, src2d, dst2d, zeros_h1, n_pad, n_chunk, h1)
    p3 = _combine_call(agg2, nd_col, ns_col, b2, W3, n_pad, bm)
    agg3 = _scatter_call(p3, src2d, dst2d, zeros_h2, n_pad, n_chunk, h2)

    z = _zcombine_call(agg3, nd_col, b3, n, 1000)
    adj = _decoder_call(z, n, 200)
    return (adj, z)


# speedup vs baseline: 1.0219x; 1.0219x over previous
import functools

import jax
import jax.numpy as jnp
from jax import lax
from jax.experimental import pallas as pl
from jax.experimental.pallas import tpu as pltpu
from jax.experimental.pallas import tpu_sc as plsc

NC = 2
NS = 16
NW = NC * NS
CH = 128

_HI = lax.Precision.HIGHEST


def _sc_mesh():
    return plsc.VectorSubcoreMesh(
        core_axis_name="c", subcore_axis_name="s", num_cores=NC, num_subcores=NS
    )


def _degree_call(src2d, dst2d, ones_v, zeros_r, n_pad, n_chunk):
    rt = n_pad // NS

    @functools.partial(
        pl.kernel,
        out_type=jax.ShapeDtypeStruct((NC, 2, n_pad), jnp.float32),
        mesh=_sc_mesh(),
        scratch_types=[
            pltpu.VMEM((n_chunk, CH), jnp.int32),
            pltpu.VMEM((n_chunk, CH), jnp.int32),
            pltpu.VMEM((CH,), jnp.float32),
            pltpu.VMEM((rt,), jnp.float32),
            pltpu.VMEM_SHARED((n_pad,), jnp.float32),
            pltpu.VMEM_SHARED((n_pad,), jnp.float32),
        ],
    )
    def k(src_h, dst_h, ones_h, zer_h, out_h, idx_s, idx_d, one_v, buf,
          deg_s, deg_d):
        c = lax.axis_index("c")
        s = lax.axis_index("s")
        wid = c * NS + s
        pltpu.sync_copy(src_h.at[pl.ds(wid * n_chunk, n_chunk)], idx_s)
        pltpu.sync_copy(dst_h.at[pl.ds(wid * n_chunk, n_chunk)], idx_d)
        pltpu.sync_copy(ones_h, one_v)
        pltpu.sync_copy(zer_h, buf)
        pltpu.sync_copy(buf, deg_s.at[pl.ds(s * rt, rt)])
        pltpu.sync_copy(buf, deg_d.at[pl.ds(s * rt, rt)])
        plsc.subcore_barrier()

        def body(j, carry):
            pltpu.sync_copy(one_v, deg_s.at[idx_s.at[j]], add=True)
            pltpu.sync_copy(one_v, deg_d.at[idx_d.at[j]], add=True)
            return carry

        lax.fori_loop(0, n_chunk, body, 0)
        plsc.subcore_barrier()
        pltpu.sync_copy(deg_s.at[pl.ds(s * rt, rt)], buf)
        pltpu.sync_copy(buf, out_h.at[c, 0, pl.ds(s * rt, rt)])
        pltpu.sync_copy(deg_d.at[pl.ds(s * rt, rt)], buf)
        pltpu.sync_copy(buf, out_h.at[c, 1, pl.ds(s * rt, rt)])

    return k(src2d, dst2d, ones_v, zeros_r)


def _scatter_call(p, src2d, dst2d, zeros_ch, n_pad, n_chunk, h):
    rt = n_pad // NS
    nz = rt // CH

    nslot = 4
    nhalf = 2
    qc = n_chunk // nhalf
    ngrp = qc // nslot

    @functools.partial(
        pl.kernel,
        out_type=jax.ShapeDtypeStruct((NC, n_pad, h), jnp.float32),
        mesh=_sc_mesh(),
        scratch_types=[
            pltpu.VMEM((qc, CH), jnp.int32),
            pltpu.VMEM((qc, CH), jnp.int32),
        ]
        + [pltpu.VMEM((CH, h), jnp.float32) for _ in range(nslot)]
        + [pltpu.VMEM_SHARED((n_pad, h), jnp.float32)]
        + [pltpu.VMEM_SHARED((n_pad, h), jnp.float32)]
        + [pltpu.SemaphoreType.DMA for _ in range(nslot)],
        compiler_params=pltpu.CompilerParams(use_tc_tiling_on_sc=False),
    )
    def k(p_h, src_h, dst_h, zer_h, out_h, idx_s, idx_d, *rest):
        rows = rest[:nslot]
        buf = rows[0]
        agg = rest[nslot]
        pspm = rest[nslot + 1]
        sems = rest[nslot + 2:]
        c = lax.axis_index("c")
        s = lax.axis_index("s")
        wid = c * NS + s
        for t in range(nz):
            pltpu.sync_copy(p_h.at[pl.ds(s * rt + t * CH, CH)], buf)
            pltpu.sync_copy(buf, pspm.at[pl.ds(s * rt + t * CH, CH)])
        pltpu.sync_copy(zer_h, buf)
        for t in range(nz):
            pltpu.sync_copy(buf, agg.at[pl.ds(s * rt + t * CH, CH)])
        plsc.subcore_barrier()

        for q in range(nhalf):
            base = wid * n_chunk + q * qc
            pltpu.sync_copy(src_h.at[pl.ds(base, qc)], idx_s)
            pltpu.sync_copy(dst_h.at[pl.ds(base, qc)], idx_d)
            for kk in range(nslot):
                pltpu.async_copy(pspm.at[idx_s.at[kk]], rows[kk], sems[kk])

            def body(g, carry):
                for kk in range(nslot):
                    j = g * nslot + kk
                    pltpu.make_async_copy(pspm.at[idx_s.at[j]], rows[kk],
                                          sems[kk]).wait()
                    pltpu.sync_copy(rows[kk], agg.at[idx_d.at[j]], add=True)

                    @pl.when(g < ngrp - 1)
                    def _():
                        pltpu.async_copy(pspm.at[idx_s.at[j + nslot]],
                                         rows[kk], sems[kk])

                return carry

            lax.fori_loop(0, ngrp, body, 0)
        plsc.subcore_barrier()
        for t in range(nz):
            pltpu.sync_copy(agg.at[pl.ds(s * rt + t * CH, CH)], buf)
            pltpu.sync_copy(buf, out_h.at[c, pl.ds(s * rt + t * CH, CH)])

    return k(p, src2d, dst2d, zeros_ch)


def _norms_kernel(deg_ref, out_ref):
    d = deg_ref[...]
    ns = lax.rsqrt(jnp.maximum(d[0, 0] + d[1, 0], 1.0))
    nd = lax.rsqrt(jnp.maximum(d[0, 1] + d[1, 1], 1.0))
    out_ref[0, :] = ns
    out_ref[1, :] = nd


def _norms_call(deg, n_pad):
    return pl.pallas_call(
        _norms_kernel,
        out_shape=jax.ShapeDtypeStruct((2, n_pad), jnp.float32),
    )(deg)


def _proj1_kernel(f_ref, w_ref, ns_ref, o_ref):
    q = lax.dot_general(f_ref[...], w_ref[...], (((1,), (0,)), ((), ())),
                        preferred_element_type=jnp.float32)
    o_ref[...] = q * ns_ref[...]


def _proj1_call(f_pad, w1, ns_col, n_pad, bm):
    d_in, h = w1.shape
    return pl.pallas_call(
        _proj1_kernel,
        grid=(n_pad // bm,),
        in_specs=[
            pl.BlockSpec((bm, d_in), lambda i: (i, 0)),
            pl.BlockSpec((d_in, h), lambda i: (0, 0)),
            pl.BlockSpec((bm, 1), lambda i: (i, 0)),
        ],
        out_specs=pl.BlockSpec((bm, h), lambda i: (i, 0)),
        out_shape=jax.ShapeDtypeStruct((n_pad, h), jnp.float32),
    )(f_pad, w1, ns_col)


def _combine_kernel(a_ref, nd_ref, ns_ref, b_ref, w_ref, o_ref):
    a = a_ref[0] + a_ref[1]
    hval = jnp.maximum(a * nd_ref[...] + b_ref[...], 0.0)
    o_ref[...] = lax.dot_general(hval * ns_ref[...], w_ref[...],
                                 (((1,), (0,)), ((), ())),
                                 preferred_element_type=jnp.float32)


def _combine_call(agg, nd_col, ns_col, b, w, n_pad, bm):
    h_in, h_out = w.shape
    grid = (n_pad // bm,)
    return pl.pallas_call(
        _combine_kernel,
        grid=grid,
        in_specs=[
            pl.BlockSpec((NC, bm, h_in), lambda i: (0, i, 0)),
            pl.BlockSpec((bm, 1), lambda i: (i, 0)),
            pl.BlockSpec((bm, 1), lambda i: (i, 0)),
            pl.BlockSpec((h_in,), lambda i: (0,)),
            pl.BlockSpec((h_in, h_out), lambda i: (0, 0)),
        ],
        out_specs=pl.BlockSpec((bm, h_out), lambda i: (i, 0)),
        out_shape=jax.ShapeDtypeStruct((n_pad, h_out), jnp.float32),
    )(agg, nd_col, ns_col, b, w)


def _zcombine_kernel(a_ref, nd_ref, b_ref, z_ref):
    z_ref[...] = (a_ref[0] + a_ref[1]) * nd_ref[...] + b_ref[...]


def _zcombine_call(agg, nd_col, b3, n, bm):
    h = agg.shape[-1]
    return pl.pallas_call(
        _zcombine_kernel,
        grid=(n // bm,),
        in_specs=[
            pl.BlockSpec((NC, bm, h), lambda i: (0, i, 0)),
            pl.BlockSpec((bm, 1), lambda i: (i, 0)),
            pl.BlockSpec((h,), lambda i: (0,)),
        ],
        out_specs=pl.BlockSpec((bm, h), lambda i: (i, 0)),
        out_shape=jax.ShapeDtypeStruct((n, h), jnp.float32),
    )(agg, nd_col, b3)


def _decoder_kernel(zi_ref, zall_ref, adj_ref):
    t = lax.dot_general(zi_ref[...], zall_ref[...], (((1,), (1,)), ((), ())),
                        preferred_element_type=jnp.float32)
    adj_ref[...] = 0.5 * jnp.tanh(0.5 * t) + 0.5


def _decoder_call(z, n, bd):
    h = z.shape[-1]
    return pl.pallas_call(
        _decoder_kernel,
        grid=(n // bd,),
        in_specs=[
            pl.BlockSpec((bd, h), lambda i: (i, 0)),
            pl.BlockSpec((n, h), lambda i: (0, 0)),
        ],
        out_specs=pl.BlockSpec((bd, n), lambda i: (i, 0)),
        out_shape=jax.ShapeDtypeStruct((n, n), jnp.float32),
    )(z, z)


def kernel(features, edge_index, W1, b1, W2, b2, W3, b3):
    n, _ = features.shape
    e = edge_index.shape[1]
    h1 = W1.shape[1]
    h2 = W3.shape[1]

    n_pad = ((n + 1 + NS * CH - 1) // (NS * CH)) * (NS * CH)
    ept = ((e + NW * CH * 8 - 1) // (NW * CH * 8)) * CH * 8
    ep = ept * NW
    n_chunk = ept // CH
    pad_row = n

    src = edge_index[0]
    dst = edge_index[1]
    pad = jnp.full((ep - e,), pad_row, jnp.int32)
    src2d = jnp.concatenate([src, pad]).reshape(ep // CH, CH)
    dst2d = jnp.concatenate([dst, pad]).reshape(ep // CH, CH)

    f_pad = jnp.pad(features, ((0, n_pad - n), (0, 0)))
    ones_v = jnp.ones((CH,), jnp.float32)
    zeros_r = jnp.zeros((n_pad // NS,), jnp.float32)
    zeros_h1 = jnp.zeros((CH, h1), jnp.float32)
    zeros_h2 = jnp.zeros((CH, h2), jnp.float32)

    bm = 2048
    deg = _degree_call(src2d, dst2d, ones_v, zeros_r, n_pad, n_chunk)
    norms = _norms_call(deg, n_pad)
    ns_col = norms[0].reshape(n_pad, 1)
    nd_col = norms[1].reshape(n_pad, 1)
    p1 = _proj1_call(f_pad, W1, ns_col, n_pad, bm)
    agg1 = _scatter_call(p1, src2d, dst2d, zeros_h1, n_pad, n_chunk, h1)
    p2 = _combine_call(agg1, nd_col, ns_col, b1, W2, n_pad, bm)
    agg2 = _scatter_call(p2, src2d, dst2d, zeros_h1, n_pad, n_chunk, h1)
    p3 = _combine_call(agg2, nd_col, ns_col, b2, W3, n_pad, bm)
    agg3 = _scatter_call(p3, src2d, dst2d, zeros_h2, n_pad, n_chunk, h2)

    z = _zcombine_call(agg3, nd_col, b3, n, 1000)
    adj = _decoder_call(z, n, 200)
    return (adj, z)


# speedup vs baseline: 1.0317x; 1.0096x over previous
import functools

import jax
import jax.numpy as jnp
from jax import lax
from jax.experimental import pallas as pl
from jax.experimental.pallas import tpu as pltpu
from jax.experimental.pallas import tpu_sc as plsc

NC = 2
NS = 16
NW = NC * NS
CH = 128

_HI = lax.Precision.HIGHEST


def _sc_mesh():
    return plsc.VectorSubcoreMesh(
        core_axis_name="c", subcore_axis_name="s", num_cores=NC, num_subcores=NS
    )


def _degree_call(src2d, dst2d, ones_v, zeros_r, n_pad, n_chunk):
    rt = n_pad // NS

    @functools.partial(
        pl.kernel,
        out_type=jax.ShapeDtypeStruct((NC, 2, n_pad), jnp.float32),
        mesh=_sc_mesh(),
        scratch_types=[
            pltpu.VMEM((n_chunk, CH), jnp.int32),
            pltpu.VMEM((n_chunk, CH), jnp.int32),
            pltpu.VMEM((CH,), jnp.float32),
            pltpu.VMEM((rt,), jnp.float32),
            pltpu.VMEM_SHARED((n_pad,), jnp.float32),
            pltpu.VMEM_SHARED((n_pad,), jnp.float32),
        ],
    )
    def k(src_h, dst_h, ones_h, zer_h, out_h, idx_s, idx_d, one_v, buf,
          deg_s, deg_d):
        c = lax.axis_index("c")
        s = lax.axis_index("s")
        wid = c * NS + s
        pltpu.sync_copy(src_h.at[pl.ds(wid * n_chunk, n_chunk)], idx_s)
        pltpu.sync_copy(dst_h.at[pl.ds(wid * n_chunk, n_chunk)], idx_d)
        pltpu.sync_copy(ones_h, one_v)
        pltpu.sync_copy(zer_h, buf)
        pltpu.sync_copy(buf, deg_s.at[pl.ds(s * rt, rt)])
        pltpu.sync_copy(buf, deg_d.at[pl.ds(s * rt, rt)])
        plsc.subcore_barrier()

        def body(j, carry):
            pltpu.sync_copy(one_v, deg_s.at[idx_s.at[j]], add=True)
            pltpu.sync_copy(one_v, deg_d.at[idx_d.at[j]], add=True)
            return carry

        lax.fori_loop(0, n_chunk, body, 0)
        plsc.subcore_barrier()
        pltpu.sync_copy(deg_s.at[pl.ds(s * rt, rt)], buf)
        pltpu.sync_copy(buf, out_h.at[c, 0, pl.ds(s * rt, rt)])
        pltpu.sync_copy(deg_d.at[pl.ds(s * rt, rt)], buf)
        pltpu.sync_copy(buf, out_h.at[c, 1, pl.ds(s * rt, rt)])

    return k(src2d, dst2d, ones_v, zeros_r)


def _scatter_call(p, src2d, dst2d, zeros_ch, n_pad, n_chunk, h):
    rt = n_pad // NS
    nz = rt // CH

    nslot = 2
    ngrp = n_chunk // nslot

    @functools.partial(
        pl.kernel,
        out_type=jax.ShapeDtypeStruct((NC, n_pad, h), jnp.float32),
        mesh=_sc_mesh(),
        scratch_types=[
            pltpu.VMEM((n_chunk, CH), jnp.int32),
            pltpu.VMEM((n_chunk, CH), jnp.int32),
        ]
        + [pltpu.VMEM((CH, h), jnp.float32) for _ in range(nslot)]
        + [pltpu.VMEM_SHARED((n_pad, h), jnp.float32)]
        + [pltpu.VMEM_SHARED((n_pad, h), jnp.float32)]
        + [pltpu.SemaphoreType.DMA for _ in range(nslot)],
        compiler_params=pltpu.CompilerParams(use_tc_tiling_on_sc=False),
    )
    def k(p_h, src_h, dst_h, zer_h, out_h, idx_s, idx_d, *rest):
        rows = rest[:nslot]
        buf = rows[0]
        agg = rest[nslot]
        pspm = rest[nslot + 1]
        sems = rest[nslot + 2:]
        c = lax.axis_index("c")
        s = lax.axis_index("s")
        wid = c * NS + s
        for t in range(nz):
            pltpu.sync_copy(p_h.at[pl.ds(s * rt + t * CH, CH)], buf)
            pltpu.sync_copy(buf, pspm.at[pl.ds(s * rt + t * CH, CH)])
        pltpu.sync_copy(zer_h, buf)
        for t in range(nz):
            pltpu.sync_copy(buf, agg.at[pl.ds(s * rt + t * CH, CH)])
        plsc.subcore_barrier()

        pltpu.sync_copy(src_h.at[pl.ds(wid * n_chunk, n_chunk)], idx_s)
        pltpu.sync_copy(dst_h.at[pl.ds(wid * n_chunk, n_chunk)], idx_d)
        for kk in range(nslot):
            pltpu.async_copy(pspm.at[idx_s.at[kk]], rows[kk], sems[kk])

        def body(g, carry):
            for kk in range(nslot):
                j = g * nslot + kk
                pltpu.make_async_copy(pspm.at[idx_s.at[j]], rows[kk],
                                      sems[kk]).wait()
                pltpu.sync_copy(rows[kk], agg.at[idx_d.at[j]], add=True)

                @pl.when(g < ngrp - 1)
                def _():
                    pltpu.async_copy(pspm.at[idx_s.at[j + nslot]],
                                     rows[kk], sems[kk])

            return carry

        lax.fori_loop(0, ngrp, body, 0)
        plsc.subcore_barrier()
        for t in range(nz):
            pltpu.sync_copy(agg.at[pl.ds(s * rt + t * CH, CH)], buf)
            pltpu.sync_copy(buf, out_h.at[c, pl.ds(s * rt + t * CH, CH)])

    return k(p, src2d, dst2d, zeros_ch)


def _norms_kernel(deg_ref, out_ref):
    d = deg_ref[...]
    ns = lax.rsqrt(jnp.maximum(d[0, 0] + d[1, 0], 1.0))
    nd = lax.rsqrt(jnp.maximum(d[0, 1] + d[1, 1], 1.0))
    out_ref[0, :] = ns
    out_ref[1, :] = nd


def _norms_call(deg, n_pad):
    return pl.pallas_call(
        _norms_kernel,
        out_shape=jax.ShapeDtypeStruct((2, n_pad), jnp.float32),
    )(deg)


def _proj1_kernel(f_ref, w_ref, ns_ref, o_ref):
    q = lax.dot_general(f_ref[...], w_ref[...], (((1,), (0,)), ((), ())),
                        preferred_element_type=jnp.float32)
    o_ref[...] = q * ns_ref[...]


def _proj1_call(f_pad, w1, ns_col, n_pad, bm):
    d_in, h = w1.shape
    return pl.pallas_call(
        _proj1_kernel,
        grid=(n_pad // bm,),
        in_specs=[
            pl.BlockSpec((bm, d_in), lambda i: (i, 0)),
            pl.BlockSpec((d_in, h), lambda i: (0, 0)),
            pl.BlockSpec((bm, 1), lambda i: (i, 0)),
        ],
        out_specs=pl.BlockSpec((bm, h), lambda i: (i, 0)),
        out_shape=jax.ShapeDtypeStruct((n_pad, h), jnp.float32),
    )(f_pad, w1, ns_col)


def _combine_kernel(a_ref, nd_ref, ns_ref, b_ref, w_ref, o_ref):
    a = a_ref[0] + a_ref[1]
    hval = jnp.maximum(a * nd_ref[...] + b_ref[...], 0.0)
    o_ref[...] = lax.dot_general(hval * ns_ref[...], w_ref[...],
                                 (((1,), (0,)), ((), ())),
                                 preferred_element_type=jnp.float32)


def _combine_call(agg, nd_col, ns_col, b, w, n_pad, bm):
    h_in, h_out = w.shape
    grid = (n_pad // bm,)
    return pl.pallas_call(
        _combine_kernel,
        grid=grid,
        in_specs=[
            pl.BlockSpec((NC, bm, h_in), lambda i: (0, i, 0)),
            pl.BlockSpec((bm, 1), lambda i: (i, 0)),
            pl.BlockSpec((bm, 1), lambda i: (i, 0)),
            pl.BlockSpec((h_in,), lambda i: (0,)),
            pl.BlockSpec((h_in, h_out), lambda i: (0, 0)),
        ],
        out_specs=pl.BlockSpec((bm, h_out), lambda i: (i, 0)),
        out_shape=jax.ShapeDtypeStruct((n_pad, h_out), jnp.float32),
    )(agg, nd_col, ns_col, b, w)


def _zcombine_kernel(a_ref, nd_ref, b_ref, z_ref):
    z_ref[...] = (a_ref[0] + a_ref[1]) * nd_ref[...] + b_ref[...]


def _zcombine_call(agg, nd_col, b3, n, bm):
    h = agg.shape[-1]
    return pl.pallas_call(
        _zcombine_kernel,
        grid=(n // bm,),
        in_specs=[
            pl.BlockSpec((NC, bm, h), lambda i: (0, i, 0)),
            pl.BlockSpec((bm, 1), lambda i: (i, 0)),
            pl.BlockSpec((h,), lambda i: (0,)),
        ],
        out_specs=pl.BlockSpec((bm, h), lambda i: (i, 0)),
        out_shape=jax.ShapeDtypeStruct((n, h), jnp.float32),
    )(agg, nd_col, b3)


def _decoder_kernel(zi_ref, zall_ref, adj_ref):
    t = lax.dot_general(zi_ref[...], zall_ref[...], (((1,), (1,)), ((), ())),
                        preferred_element_type=jnp.float32)
    adj_ref[...] = 0.5 * jnp.tanh(0.5 * t) + 0.5


def _decoder_call(z, n, bd):
    h = z.shape[-1]
    return pl.pallas_call(
        _decoder_kernel,
        grid=(n // bd,),
        in_specs=[
            pl.BlockSpec((bd, h), lambda i: (i, 0)),
            pl.BlockSpec((n, h), lambda i: (0, 0)),
        ],
        out_specs=pl.BlockSpec((bd, n), lambda i: (i, 0)),
        out_shape=jax.ShapeDtypeStruct((n, n), jnp.float32),
    )(z, z)


def kernel(features, edge_index, W1, b1, W2, b2, W3, b3):
    n, _ = features.shape
    e = edge_index.shape[1]
    h1 = W1.shape[1]
    h2 = W3.shape[1]

    n_pad = ((n + 1 + NS * CH - 1) // (NS * CH)) * (NS * CH)
    ept = ((e + NW * CH * 8 - 1) // (NW * CH * 8)) * CH * 8
    ep = ept * NW
    n_chunk = ept // CH
    pad_row = n

    src = edge_index[0]
    dst = edge_index[1]
    pad = jnp.full((ep - e,), pad_row, jnp.int32)
    src2d = jnp.concatenate([src, pad]).reshape(ep // CH, CH)
    dst2d = jnp.concatenate([dst, pad]).reshape(ep // CH, CH)

    f_pad = jnp.pad(features, ((0, n_pad - n), (0, 0)))
    ones_v = jnp.ones((CH,), jnp.float32)
    zeros_r = jnp.zeros((n_pad // NS,), jnp.float32)
    zeros_h1 = jnp.zeros((CH, h1), jnp.float32)
    zeros_h2 = jnp.zeros((CH, h2), jnp.float32)

    bm = 2048
    deg = _degree_call(src2d, dst2d, ones_v, zeros_r, n_pad, n_chunk)
    norms = _norms_call(deg, n_pad)
    ns_col = norms[0].reshape(n_pad, 1)
    nd_col = norms[1].reshape(n_pad, 1)
    p1 = _proj1_call(f_pad, W1, ns_col, n_pad, bm)
    agg1 = _scatter_call(p1, src2d, dst2d, zeros_h1, n_pad, n_chunk, h1)
    p2 = _combine_call(agg1, nd_col, ns_col, b1, W2, n_pad, bm)
    agg2 = _scatter_call(p2, src2d, dst2d, zeros_h1, n_pad, n_chunk, h1)
    p3 = _combine_call(agg2, nd_col, ns_col, b2, W3, n_pad, bm)
    agg3 = _scatter_call(p3, src2d, dst2d, zeros_h2, n_pad, n_chunk, h2)

    z = _zcombine_call(agg3, nd_col, b3, n, 1000)
    adj = _decoder_call(z, n, 200)
    return (adj, z)


# speedup vs baseline: 1.0572x; 1.0247x over previous
import functools

import jax
import jax.numpy as jnp
from jax import lax
from jax.experimental import pallas as pl
from jax.experimental.pallas import tpu as pltpu
from jax.experimental.pallas import tpu_sc as plsc

NC = 2
NS = 16
NW = NC * NS
CH = 128

_HI = lax.Precision.HIGHEST


def _sc_mesh():
    return plsc.VectorSubcoreMesh(
        core_axis_name="c", subcore_axis_name="s", num_cores=NC, num_subcores=NS
    )


def _degree_call(src2d, dst2d, ones_v, zeros_r, n_pad, n_chunk):
    rt = n_pad // NS

    @functools.partial(
        pl.kernel,
        out_type=jax.ShapeDtypeStruct((NC, 2, n_pad), jnp.float32),
        mesh=_sc_mesh(),
        scratch_types=[
            pltpu.VMEM((n_chunk, CH), jnp.int32),
            pltpu.VMEM((n_chunk, CH), jnp.int32),
            pltpu.VMEM((CH,), jnp.float32),
            pltpu.VMEM((rt,), jnp.float32),
            pltpu.VMEM_SHARED((n_pad,), jnp.float32),
            pltpu.VMEM_SHARED((n_pad,), jnp.float32),
        ],
    )
    def k(src_h, dst_h, ones_h, zer_h, out_h, idx_s, idx_d, one_v, buf,
          deg_s, deg_d):
        c = lax.axis_index("c")
        s = lax.axis_index("s")
        wid = c * NS + s
        pltpu.sync_copy(src_h.at[pl.ds(wid * n_chunk, n_chunk)], idx_s)
        pltpu.sync_copy(dst_h.at[pl.ds(wid * n_chunk, n_chunk)], idx_d)
        pltpu.sync_copy(ones_h, one_v)
        pltpu.sync_copy(zer_h, buf)
        pltpu.sync_copy(buf, deg_s.at[pl.ds(s * rt, rt)])
        pltpu.sync_copy(buf, deg_d.at[pl.ds(s * rt, rt)])
        plsc.subcore_barrier()

        def body(j, carry):
            pltpu.sync_copy(one_v, deg_s.at[idx_s.at[j]], add=True)
            pltpu.sync_copy(one_v, deg_d.at[idx_d.at[j]], add=True)
            return carry

        lax.fori_loop(0, n_chunk, body, 0)
        plsc.subcore_barrier()
        pltpu.sync_copy(deg_s.at[pl.ds(s * rt, rt)], buf)
        pltpu.sync_copy(buf, out_h.at[c, 0, pl.ds(s * rt, rt)])
        pltpu.sync_copy(deg_d.at[pl.ds(s * rt, rt)], buf)
        pltpu.sync_copy(buf, out_h.at[c, 1, pl.ds(s * rt, rt)])

    return k(src2d, dst2d, ones_v, zeros_r)


def _scatter_call(p, src2d, dst2d, zeros_ch, n_pad, n_chunk, h):
    rt = n_pad // NS
    nz = rt // CH

    nslot = 2
    ngrp = n_chunk // nslot

    @functools.partial(
        pl.kernel,
        out_type=jax.ShapeDtypeStruct((NC, n_pad, h), jnp.float32),
        mesh=_sc_mesh(),
        scratch_types=[
            pltpu.VMEM((n_chunk, CH), jnp.int32),
            pltpu.VMEM((n_chunk, CH), jnp.int32),
        ]
        + [pltpu.VMEM((CH, h), jnp.float32) for _ in range(nslot)]
        + [pltpu.VMEM_SHARED((n_pad, h), jnp.float32)]
        + [pltpu.VMEM_SHARED((n_pad, h), jnp.float32)]
        + [pltpu.SemaphoreType.DMA for _ in range(nslot)],
        compiler_params=pltpu.CompilerParams(use_tc_tiling_on_sc=False),
    )
    def k(p_h, src_h, dst_h, zer_h, out_h, idx_s, idx_d, *rest):
        rows = rest[:nslot]
        buf = rows[0]
        agg = rest[nslot]
        pspm = rest[nslot + 1]
        sems = rest[nslot + 2:]
        c = lax.axis_index("c")
        s = lax.axis_index("s")
        wid = c * NS + s
        pltpu.sync_copy(p_h.at[pl.ds(s * rt, rt)], pspm.at[pl.ds(s * rt, rt)])
        pltpu.sync_copy(zer_h, buf)
        for t in range(nz):
            pltpu.sync_copy(buf, agg.at[pl.ds(s * rt + t * CH, CH)])
        plsc.subcore_barrier()

        pltpu.sync_copy(src_h.at[pl.ds(wid * n_chunk, n_chunk)], idx_s)
        pltpu.sync_copy(dst_h.at[pl.ds(wid * n_chunk, n_chunk)], idx_d)
        for kk in range(nslot):
            pltpu.async_copy(pspm.at[idx_s.at[kk]], rows[kk], sems[kk])

        def body(g, carry):
            for kk in range(nslot):
                j = g * nslot + kk
                pltpu.make_async_copy(pspm.at[idx_s.at[j]], rows[kk],
                                      sems[kk]).wait()
                pltpu.sync_copy(rows[kk], agg.at[idx_d.at[j]], add=True)

                @pl.when(g < ngrp - 1)
                def _():
                    pltpu.async_copy(pspm.at[idx_s.at[j + nslot]],
                                     rows[kk], sems[kk])

            return carry

        lax.fori_loop(0, ngrp, body, 0)
        plsc.subcore_barrier()
        pltpu.sync_copy(agg.at[pl.ds(s * rt, rt)],
                        out_h.at[c, pl.ds(s * rt, rt)])

    return k(p, src2d, dst2d, zeros_ch)


def _norms_kernel(deg_ref, out_ref):
    d = deg_ref[...]
    ns = lax.rsqrt(jnp.maximum(d[0, 0] + d[1, 0], 1.0))
    nd = lax.rsqrt(jnp.maximum(d[0, 1] + d[1, 1], 1.0))
    out_ref[0, :] = ns
    out_ref[1, :] = nd


def _norms_call(deg, n_pad):
    return pl.pallas_call(
        _norms_kernel,
        out_shape=jax.ShapeDtypeStruct((2, n_pad), jnp.float32),
    )(deg)


def _proj1_kernel(f_ref, w_ref, ns_ref, o_ref):
    q = lax.dot_general(f_ref[...], w_ref[...], (((1,), (0,)), ((), ())),
                        preferred_element_type=jnp.float32)
    o_ref[...] = q * ns_ref[...]


def _proj1_call(f_pad, w1, ns_col, n_pad, bm):
    d_in, h = w1.shape
    return pl.pallas_call(
        _proj1_kernel,
        grid=(n_pad // bm,),
        in_specs=[
            pl.BlockSpec((bm, d_in), lambda i: (i, 0)),
            pl.BlockSpec((d_in, h), lambda i: (0, 0)),
            pl.BlockSpec((bm, 1), lambda i: (i, 0)),
        ],
        out_specs=pl.BlockSpec((bm, h), lambda i: (i, 0)),
        out_shape=jax.ShapeDtypeStruct((n_pad, h), jnp.float32),
    )(f_pad, w1, ns_col)


def _combine_kernel(a_ref, nd_ref, ns_ref, b_ref, w_ref, o_ref):
    a = a_ref[0] + a_ref[1]
    hval = jnp.maximum(a * nd_ref[...] + b_ref[...], 0.0)
    o_ref[...] = lax.dot_general(hval * ns_ref[...], w_ref[...],
                                 (((1,), (0,)), ((), ())),
                                 preferred_element_type=jnp.float32)


def _combine_call(agg, nd_col, ns_col, b, w, n_pad, bm):
    h_in, h_out = w.shape
    grid = (n_pad // bm,)
    return pl.pallas_call(
        _combine_kernel,
        grid=grid,
        in_specs=[
            pl.BlockSpec((NC, bm, h_in), lambda i: (0, i, 0)),
            pl.BlockSpec((bm, 1), lambda i: (i, 0)),
            pl.BlockSpec((bm, 1), lambda i: (i, 0)),
            pl.BlockSpec((h_in,), lambda i: (0,)),
            pl.BlockSpec((h_in, h_out), lambda i: (0, 0)),
        ],
        out_specs=pl.BlockSpec((bm, h_out), lambda i: (i, 0)),
        out_shape=jax.ShapeDtypeStruct((n_pad, h_out), jnp.float32),
    )(agg, nd_col, ns_col, b, w)


def _zcombine_kernel(a_ref, nd_ref, b_ref, z_ref):
    z_ref[...] = (a_ref[0] + a_ref[1]) * nd_ref[...] + b_ref[...]


def _zcombine_call(agg, nd_col, b3, n, bm):
    h = agg.shape[-1]
    return pl.pallas_call(
        _zcombine_kernel,
        grid=(n // bm,),
        in_specs=[
            pl.BlockSpec((NC, bm, h), lambda i: (0, i, 0)),
            pl.BlockSpec((bm, 1), lambda i: (i, 0)),
            pl.BlockSpec((h,), lambda i: (0,)),
        ],
        out_specs=pl.BlockSpec((bm, h), lambda i: (i, 0)),
        out_shape=jax.ShapeDtypeStruct((n, h), jnp.float32),
    )(agg, nd_col, b3)


def _decoder_kernel(zi_ref, zall_ref, adj_ref):
    t = lax.dot_general(zi_ref[...], zall_ref[...], (((1,), (1,)), ((), ())),
                        preferred_element_type=jnp.float32)
    adj_ref[...] = 0.5 * jnp.tanh(0.5 * t) + 0.5


def _decoder_call(z, n, bd):
    h = z.shape[-1]
    return pl.pallas_call(
        _decoder_kernel,
        grid=(n // bd,),
        in_specs=[
            pl.BlockSpec((bd, h), lambda i: (i, 0)),
            pl.BlockSpec((n, h), lambda i: (0, 0)),
        ],
        out_specs=pl.BlockSpec((bd, n), lambda i: (i, 0)),
        out_shape=jax.ShapeDtypeStruct((n, n), jnp.float32),
    )(z, z)


def kernel(features, edge_index, W1, b1, W2, b2, W3, b3):
    n, _ = features.shape
    e = edge_index.shape[1]
    h1 = W1.shape[1]
    h2 = W3.shape[1]

    n_pad = ((n + 1 + NS * CH - 1) // (NS * CH)) * (NS * CH)
    ept = ((e + NW * CH * 8 - 1) // (NW * CH * 8)) * CH * 8
    ep = ept * NW
    n_chunk = ept // CH
    pad_row = n

    src = edge_index[0]
    dst = edge_index[1]
    pad = jnp.full((ep - e,), pad_row, jnp.int32)
    src2d = jnp.concatenate([src, pad]).reshape(ep // CH, CH)
    dst2d = jnp.concatenate([dst, pad]).reshape(ep // CH, CH)

    f_pad = jnp.pad(features, ((0, n_pad - n), (0, 0)))
    ones_v = jnp.ones((CH,), jnp.float32)
    zeros_r = jnp.zeros((n_pad // NS,), jnp.float32)
    zeros_h1 = jnp.zeros((CH, h1), jnp.float32)
    zeros_h2 = jnp.zeros((CH, h2), jnp.float32)

    bm = 2048
    deg = _degree_call(src2d, dst2d, ones_v, zeros_r, n_pad, n_chunk)
    norms = _norms_call(deg, n_pad)
    ns_col = norms[0].reshape(n_pad, 1)
    nd_col = norms[1].reshape(n_pad, 1)
    p1 = _proj1_call(f_pad, W1, ns_col, n_pad, bm)
    agg1 = _scatter_call(p1, src2d, dst2d, zeros_h1, n_pad, n_chunk, h1)
    p2 = _combine_call(agg1, nd_col, ns_col, b1, W2, n_pad, bm)
    agg2 = _scatter_call(p2, src2d, dst2d, zeros_h1, n_pad, n_chunk, h1)
    p3 = _combine_call(agg2, nd_col, ns_col, b2, W3, n_pad, bm)
    agg3 = _scatter_call(p3, src2d, dst2d, zeros_h2, n_pad, n_chunk, h2)

    z = _zcombine_call(agg3, nd_col, b3, n, 1000)
    adj = _decoder_call(z, n, 200)
    return (adj, z)


# speedup vs baseline: 1.0673x; 1.0096x over previous
import functools

import jax
import jax.numpy as jnp
from jax import lax
from jax.experimental import pallas as pl
from jax.experimental.pallas import tpu as pltpu
from jax.experimental.pallas import tpu_sc as plsc

NC = 2
NS = 16
NW = NC * NS
CH = 128

_HI = lax.Precision.HIGHEST


def _sc_mesh():
    return plsc.VectorSubcoreMesh(
        core_axis_name="c", subcore_axis_name="s", num_cores=NC, num_subcores=NS
    )


def _degree_call(src2d, dst2d, ones_v, zeros_r, n_pad, n_chunk):
    rt = n_pad // NS

    @functools.partial(
        pl.kernel,
        out_type=jax.ShapeDtypeStruct((NC, 2, n_pad), jnp.float32),
        mesh=_sc_mesh(),
        scratch_types=[
            pltpu.VMEM((n_chunk, CH), jnp.int32),
            pltpu.VMEM((n_chunk, CH), jnp.int32),
            pltpu.VMEM((CH,), jnp.float32),
            pltpu.VMEM((rt,), jnp.float32),
            pltpu.VMEM_SHARED((n_pad,), jnp.float32),
            pltpu.VMEM_SHARED((n_pad,), jnp.float32),
        ],
    )
    def k(src_h, dst_h, ones_h, zer_h, out_h, idx_s, idx_d, one_v, buf,
          deg_s, deg_d):
        c = lax.axis_index("c")
        s = lax.axis_index("s")
        wid = c * NS + s
        pltpu.sync_copy(src_h.at[pl.ds(wid * n_chunk, n_chunk)], idx_s)
        pltpu.sync_copy(dst_h.at[pl.ds(wid * n_chunk, n_chunk)], idx_d)
        pltpu.sync_copy(ones_h, one_v)
        pltpu.sync_copy(zer_h, buf)
        pltpu.sync_copy(buf, deg_s.at[pl.ds(s * rt, rt)])
        pltpu.sync_copy(buf, deg_d.at[pl.ds(s * rt, rt)])
        plsc.subcore_barrier()

        def body(j, carry):
            pltpu.sync_copy(one_v, deg_s.at[idx_s.at[j]], add=True)
            pltpu.sync_copy(one_v, deg_d.at[idx_d.at[j]], add=True)
            return carry

        lax.fori_loop(0, n_chunk, body, 0)
        plsc.subcore_barrier()
        pltpu.sync_copy(deg_s.at[pl.ds(s * rt, rt)], buf)
        pltpu.sync_copy(buf, out_h.at[c, 0, pl.ds(s * rt, rt)])
        pltpu.sync_copy(deg_d.at[pl.ds(s * rt, rt)], buf)
        pltpu.sync_copy(buf, out_h.at[c, 1, pl.ds(s * rt, rt)])

    return k(src2d, dst2d, ones_v, zeros_r)


def _scatter_call(p, src2d, dst2d, zeros_ch, n_pad, n_chunk, h):
    rt = n_pad // NS
    nz = rt // CH

    nslot = 2
    ngrp = n_chunk // nslot

    @functools.partial(
        pl.kernel,
        out_type=jax.ShapeDtypeStruct((NC, n_pad, h), jnp.float32),
        mesh=_sc_mesh(),
        scratch_types=[
            pltpu.VMEM((n_chunk, CH), jnp.int32),
            pltpu.VMEM((n_chunk, CH), jnp.int32),
        ]
        + [pltpu.VMEM((CH, h), jnp.float32) for _ in range(nslot)]
        + [pltpu.VMEM_SHARED((n_pad, h), jnp.float32)]
        + [pltpu.VMEM_SHARED((n_pad, h), jnp.float32)]
        + [pltpu.SemaphoreType.DMA for _ in range(nslot)],
        compiler_params=pltpu.CompilerParams(use_tc_tiling_on_sc=False),
    )
    def k(p_h, src_h, dst_h, zer_h, out_h, idx_s, idx_d, *rest):
        rows = rest[:nslot]
        buf = rows[0]
        agg = rest[nslot]
        pspm = rest[nslot + 1]
        sems = rest[nslot + 2:]
        c = lax.axis_index("c")
        s = lax.axis_index("s")
        wid = c * NS + s
        pltpu.sync_copy(p_h.at[pl.ds(s * rt, rt)], pspm.at[pl.ds(s * rt, rt)])
        pltpu.sync_copy(zer_h, buf)
        for t in range(nz):
            pltpu.sync_copy(buf, agg.at[pl.ds(s * rt + t * CH, CH)])
        plsc.subcore_barrier()

        pltpu.sync_copy(src_h.at[pl.ds(wid * n_chunk, n_chunk)], idx_s)
        pltpu.sync_copy(dst_h.at[pl.ds(wid * n_chunk, n_chunk)], idx_d)
        for kk in range(nslot):
            pltpu.async_copy(pspm.at[idx_s.at[kk]], rows[kk], sems[kk])

        def body(g, carry):
            for kk in range(nslot):
                j = g * nslot + kk
                pltpu.make_async_copy(pspm.at[idx_s.at[j]], rows[kk],
                                      sems[kk]).wait()
                pltpu.sync_copy(rows[kk], agg.at[idx_d.at[j]], add=True)

                @pl.when(g < ngrp - 1)
                def _():
                    pltpu.async_copy(pspm.at[idx_s.at[j + nslot]],
                                     rows[kk], sems[kk])

            return carry

        lax.fori_loop(0, ngrp, body, 0)
        plsc.subcore_barrier()
        pltpu.sync_copy(agg.at[pl.ds(s * rt, rt)],
                        out_h.at[c, pl.ds(s * rt, rt)])

    return k(p, src2d, dst2d, zeros_ch)


def _proj_kernel(f_ref, w_ref, o_ref):
    o_ref[...] = lax.dot_general(f_ref[...], w_ref[...],
                                 (((1,), (0,)), ((), ())),
                                 preferred_element_type=jnp.float32)


def _proj_call(f_pad, w1, n_pad, bm):
    d_in, h = w1.shape
    return pl.pallas_call(
        _proj_kernel,
        grid=(n_pad // bm,),
        in_specs=[
            pl.BlockSpec((bm, d_in), lambda i: (i, 0)),
            pl.BlockSpec((d_in, h), lambda i: (0, 0)),
        ],
        out_specs=pl.BlockSpec((bm, h), lambda i: (i, 0)),
        out_shape=jax.ShapeDtypeStruct((n_pad, h), jnp.float32),
    )(f_pad, w1)


def _normscale_kernel(deg_ref, q_ref, p_ref, ns_ref, nd_ref):
    d = deg_ref[...]
    ns = lax.rsqrt(jnp.maximum(d[0, 0] + d[1, 0], 1.0))[:, None]
    nd = lax.rsqrt(jnp.maximum(d[0, 1] + d[1, 1], 1.0))[:, None]
    ns_ref[...] = ns
    nd_ref[...] = nd
    p_ref[...] = q_ref[...] * ns


def _normscale_call(deg, q1, n_pad, bm):
    h = q1.shape[-1]
    return pl.pallas_call(
        _normscale_kernel,
        grid=(n_pad // bm,),
        in_specs=[
            pl.BlockSpec((NC, 2, bm), lambda i: (0, 0, i)),
            pl.BlockSpec((bm, h), lambda i: (i, 0)),
        ],
        out_specs=[
            pl.BlockSpec((bm, h), lambda i: (i, 0)),
            pl.BlockSpec((bm, 1), lambda i: (i, 0)),
            pl.BlockSpec((bm, 1), lambda i: (i, 0)),
        ],
        out_shape=[
            jax.ShapeDtypeStruct((n_pad, h), jnp.float32),
            jax.ShapeDtypeStruct((n_pad, 1), jnp.float32),
            jax.ShapeDtypeStruct((n_pad, 1), jnp.float32),
        ],
    )(deg, q1)


def _combine_kernel(a_ref, nd_ref, ns_ref, b_ref, w_ref, o_ref):
    a = a_ref[0] + a_ref[1]
    hval = jnp.maximum(a * nd_ref[...] + b_ref[...], 0.0)
    o_ref[...] = lax.dot_general(hval * ns_ref[...], w_ref[...],
                                 (((1,), (0,)), ((), ())),
                                 preferred_element_type=jnp.float32)


def _combine_call(agg, nd_col, ns_col, b, w, n_pad, bm):
    h_in, h_out = w.shape
    grid = (n_pad // bm,)
    return pl.pallas_call(
        _combine_kernel,
        grid=grid,
        in_specs=[
            pl.BlockSpec((NC, bm, h_in), lambda i: (0, i, 0)),
            pl.BlockSpec((bm, 1), lambda i: (i, 0)),
            pl.BlockSpec((bm, 1), lambda i: (i, 0)),
            pl.BlockSpec((h_in,), lambda i: (0,)),
            pl.BlockSpec((h_in, h_out), lambda i: (0, 0)),
        ],
        out_specs=pl.BlockSpec((bm, h_out), lambda i: (i, 0)),
        out_shape=jax.ShapeDtypeStruct((n_pad, h_out), jnp.float32),
    )(agg, nd_col, ns_col, b, w)


def _zcombine_kernel(a_ref, nd_ref, b_ref, z_ref):
    z_ref[...] = (a_ref[0] + a_ref[1]) * nd_ref[...] + b_ref[...]


def _zcombine_call(agg, nd_col, b3, n, bm):
    h = agg.shape[-1]
    return pl.pallas_call(
        _zcombine_kernel,
        grid=(n // bm,),
        in_specs=[
            pl.BlockSpec((NC, bm, h), lambda i: (0, i, 0)),
            pl.BlockSpec((bm, 1), lambda i: (i, 0)),
            pl.BlockSpec((h,), lambda i: (0,)),
        ],
        out_specs=pl.BlockSpec((bm, h), lambda i: (i, 0)),
        out_shape=jax.ShapeDtypeStruct((n, h), jnp.float32),
    )(agg, nd_col, b3)


def _decoder_kernel(zi_ref, zall_ref, adj_ref):
    t = lax.dot_general(zi_ref[...], zall_ref[...], (((1,), (1,)), ((), ())),
                        preferred_element_type=jnp.float32)
    adj_ref[...] = 0.5 * jnp.tanh(0.5 * t) + 0.5


def _decoder_call(z, n, bd):
    h = z.shape[-1]
    return pl.pallas_call(
        _decoder_kernel,
        grid=(n // bd,),
        in_specs=[
            pl.BlockSpec((bd, h), lambda i: (i, 0)),
            pl.BlockSpec((n, h), lambda i: (0, 0)),
        ],
        out_specs=pl.BlockSpec((bd, n), lambda i: (i, 0)),
        out_shape=jax.ShapeDtypeStruct((n, n), jnp.float32),
    )(z, z)


def kernel(features, edge_index, W1, b1, W2, b2, W3, b3):
    n, _ = features.shape
    e = edge_index.shape[1]
    h1 = W1.shape[1]
    h2 = W3.shape[1]

    n_pad = ((n + 1 + NS * CH - 1) // (NS * CH)) * (NS * CH)
    ept = ((e + NW * CH * 8 - 1) // (NW * CH * 8)) * CH * 8
    ep = ept * NW
    n_chunk = ept // CH
    pad_row = n

    src = edge_index[0]
    dst = edge_index[1]
    pad = jnp.full((ep - e,), pad_row, jnp.int32)
    src2d = jnp.concatenate([src, pad]).reshape(ep // CH, CH)
    dst2d = jnp.concatenate([dst, pad]).reshape(ep // CH, CH)

    f_pad = jnp.pad(features, ((0, n_pad - n), (0, 0)))
    ones_v = jnp.ones((CH,), jnp.float32)
    zeros_r = jnp.zeros((n_pad // NS,), jnp.float32)
    zeros_h1 = jnp.zeros((CH, h1), jnp.float32)
    zeros_h2 = jnp.zeros((CH, h2), jnp.float32)

    bm = 2048
    deg = _degree_call(src2d, dst2d, ones_v, zeros_r, n_pad, n_chunk)
    q1 = _proj_call(f_pad, W1, n_pad, bm)
    p1, ns_col, nd_col = _normscale_call(deg, q1, n_pad, bm)
    agg1 = _scatter_call(p1, src2d, dst2d, zeros_h1, n_pad, n_chunk, h1)
    p2 = _combine_call(agg1, nd_col, ns_col, b1, W2, n_pad, bm)
    agg2 = _scatter_call(p2, src2d, dst2d, zeros_h1, n_pad, n_chunk, h1)
    p3 = _combine_call(agg2, nd_col, ns_col, b2, W3, n_pad, bm)
    agg3 = _scatter_call(p3, src2d, dst2d, zeros_h2, n_pad, n_chunk, h2)

    z = _zcombine_call(agg3, nd_col, b3, n, 1000)
    adj = _decoder_call(z, n, 200)
    return (adj, z)
